# baseline (speedup 1.0000x reference)
import functools

import jax
import jax.numpy as jnp
import numpy as np
from jax import lax
from jax.experimental import pallas as pl
from jax.experimental.pallas import tpu as pltpu
from jax.experimental.pallas import tpu_sc as plsc

F = 26
D = 128
B = 16384
NP = F * (F - 1) // 2
DIN = F * D

_NC, _NS = 2, 16
_NW = _NC * _NS
_CH = 128

_BLK = 512
_NSPLIT = 8
_NB = B // _NSPLIT


def _make_sc_body(per_w, nchunk):
  def _sc_body(t2_h, t1_h, idx2_h, idxb2_h, e2_h, e1_h,
               idx2_v, idxb_v, t1_v, rowbuf, valbuf, gsem, psem, vsem):
    wid = lax.axis_index("s") * _NC + lax.axis_index("c")
    base = wid * per_w

    pltpu.sync_copy(idx2_h.at[wid], idx2_v)
    pltpu.sync_copy(idxb2_h.at[wid], idxb_v)

    pltpu.sync_copy(t1_h, t1_v)

    def vbody(c, carry):
        row = idxb_v.at[c]
        for j in range(_CH // 16):
            iv = row[pl.ds(j * 16, 16)]
            valbuf[pl.ds(c * _CH + j * 16, 16)] = plsc.load_gather(t1_v, [iv])
        return carry

    lax.fori_loop(0, nchunk, vbody, 0)
    vput = pltpu.async_copy(valbuf, e1_h.at[pl.ds(base, per_w)], vsem)

    def start_gather(c):
        return pltpu.async_copy(
            t2_h.at[idx2_v.at[c]], rowbuf.at[c % 2], gsem.at[c % 2])

    def start_put(c):
        return pltpu.async_copy(
            rowbuf.at[c % 2], e2_h.at[pl.ds(base + c * _CH, _CH)],
            psem.at[c % 2])

    g_h = [None, None]
    put_h = [None, None]
    g_h[0] = start_gather(0)
    for c in range(nchunk):
        nxt = c + 1
        if nxt < nchunk:
            if put_h[nxt % 2] is not None:
                put_h[nxt % 2].wait()
            g_h[nxt % 2] = start_gather(nxt)
        g_h[c % 2].wait()
        put_h[c % 2] = start_put(c)
    put_h[(nchunk - 1) % 2].wait()
    put_h[(nchunk - 2) % 2].wait()
    vput.wait()

  return _sc_body


@functools.lru_cache(maxsize=2)
def _sc_gather(nb):
  bf = nb * F
  per_w = bf // _NW
  nchunk = per_w // _CH
  return pl.kernel(
    _make_sc_body(per_w, nchunk),
    out_type=(
        jax.ShapeDtypeStruct((bf, D), jnp.float32),
        jax.ShapeDtypeStruct((bf,), jnp.float32),
    ),
    mesh=plsc.VectorSubcoreMesh(core_axis_name="c", subcore_axis_name="s",
                                num_cores=_NC, num_subcores=_NS),
    scratch_types=[
        pltpu.VMEM((nchunk, _CH), jnp.int32),
        pltpu.VMEM((nchunk, _CH), jnp.int32),
        pltpu.VMEM((26000,), jnp.float32),
        pltpu.VMEM((2, _CH, D), jnp.float32),
        pltpu.VMEM((per_w,), jnp.float32),
        pltpu.SemaphoreType.DMA((2,)),
        pltpu.SemaphoreType.DMA((2,)),
        pltpu.SemaphoreType.DMA,
    ],
    compiler_params=pltpu.CompilerParams(needs_layout_passes=False),
  )


def _tc_body(E_ref, e1_ref, W1_ref, b1_ref, W2_ref, b2_ref, W3_ref, b3_ref,
             S_ref, wh_ref, sc_ref, out_ref):
    Es = [E_ref[f].astype(jnp.bfloat16) for f in range(F)]

    acc = b1_ref[...][None, :]
    for f in range(F):
        acc = acc + lax.dot_general(
            Es[f], W1_ref[f],
            (((1,), (0,)), ((), ())), preferred_element_type=jnp.float32)
    h = jnp.maximum(acc, 0.0)
    h = jnp.maximum(
        lax.dot_general(h, W2_ref[...], (((1,), (0,)), ((), ())),
                        preferred_element_type=jnp.float32)
        + b2_ref[...][None, :], 0.0)
    h = jnp.maximum(
        lax.dot_general(h, W3_ref[...], (((1,), (0,)), ((), ())),
                        preferred_element_type=jnp.float32)
        + b3_ref[...][None, :], 0.0)
    deep = lax.dot_general(h, wh_ref[...], (((1,), (0,)), ((), ())),
                           preferred_element_type=jnp.float32)

    M = None
    for f in range(F - 1):
        r = None
        for g in range(f + 1, F):
            t = S_ref[f, g].astype(jnp.bfloat16) * Es[g]
            r = t if r is None else r + t
        m = Es[f] * r
        M = m if M is None else M + m
    pair = jnp.sum(M.astype(jnp.float32), axis=1, keepdims=True)

    first = jnp.sum(e1_ref[...], axis=1, keepdims=True)
    logit = sc_ref[0] * first + pair + deep + sc_ref[1]
    out_ref[...] = jax.nn.sigmoid(logit)


@functools.lru_cache(maxsize=2)
def _tc_call(nb):
  return pl.pallas_call(
    _tc_body,
    grid=(nb // _BLK,),
    in_specs=[
        pl.BlockSpec((F, _BLK, D), lambda i: (0, i, 0)),
        pl.BlockSpec((_BLK, F), lambda i: (i, 0)),
        pl.BlockSpec((F, D, 128), lambda i: (0, 0, 0)),
        pl.BlockSpec((128,), lambda i: (0,)),
        pl.BlockSpec((128, 64), lambda i: (0, 0)),
        pl.BlockSpec((64,), lambda i: (0,)),
        pl.BlockSpec((64, 32), lambda i: (0, 0)),
        pl.BlockSpec((32,), lambda i: (0,)),
        pl.BlockSpec(memory_space=pltpu.SMEM),
        pl.BlockSpec((32, 1), lambda i: (0, 0)),
        pl.BlockSpec(memory_space=pltpu.SMEM),
    ],
    out_specs=pl.BlockSpec((_BLK, 1), lambda i: (i, 0)),
    out_shape=jax.ShapeDtypeStruct((nb, 1), jnp.float32),
  )


_OFFSETS = np.concatenate([[0], np.cumsum([1000] * 26)[:-1]]).astype(np.int32)
_IU, _JU = np.triu_indices(F, k=1)


def kernel(x, table1, table2, W1, b1, W2, b2, W3, b3, fcW, fcb):
    idx = x + jnp.asarray(_OFFSETS)[None, :]
    idxT = idx.T
    t1f = table1.reshape(-1)
    S = jnp.zeros((F, F), jnp.float32).at[_IU, _JU].set(fcW[1:1 + NP, 0])
    wh = fcW[1 + NP:, :]
    sc = jnp.concatenate([fcW[0], fcb])
    W1r = W1.astype(jnp.bfloat16).reshape(F, D, 128)

    gathered = []
    nchunk = F * _NB // _CH // _NW
    for s in range(_NSPLIT):
        lo = s * _NB
        idxfm3 = idxT[:, lo:lo + _NB].reshape(_NW, nchunk, _CH)
        idxb3 = idx[lo:lo + _NB].reshape(_NW, nchunk, _CH)
        gathered.append(_sc_gather(_NB)(table2, t1f, idxfm3, idxb3))
    outs = []
    for e2, e1v in gathered:
        outs.append(_tc_call(_NB)(
            e2.reshape(F, _NB, D), e1v.reshape(_NB, F),
            W1r, b1, W2, b2, W3, b3, S, wh, sc))
    return jnp.concatenate(outs, axis=0)


# speedup vs baseline: 1.0621x; 1.0621x over previous
import functools

import jax
import jax.numpy as jnp
import numpy as np
from jax import lax
from jax.experimental import pallas as pl
from jax.experimental.pallas import tpu as pltpu
from jax.experimental.pallas import tpu_sc as plsc

F = 26
D = 128
B = 16384
NP = F * (F - 1) // 2
DIN = F * D

_NC, _NS = 2, 16
_NW = _NC * _NS
_CH = 128

_BLK = 1024
_NSPLIT = 4
_NB = B // _NSPLIT


def _make_sc_body(per_w, nchunk):
  def _sc_body(t2_h, t1_h, idx2_h, idxb2_h, e2_h, e1_h,
               idx2_v, idxb_v, t1_v, rowbuf, valbuf, gsem, psem, vsem):
    wid = lax.axis_index("s") * _NC + lax.axis_index("c")
    base = wid * per_w

    pltpu.sync_copy(idx2_h.at[wid], idx2_v)
    pltpu.sync_copy(idxb2_h.at[wid], idxb_v)

    pltpu.sync_copy(t1_h, t1_v)

    def vbody(c, carry):
        row = idxb_v.at[c]
        for j in range(_CH // 16):
            iv = row[pl.ds(j * 16, 16)]
            valbuf[pl.ds(c * _CH + j * 16, 16)] = plsc.load_gather(t1_v, [iv])
        return carry

    lax.fori_loop(0, nchunk, vbody, 0)
    vput = pltpu.async_copy(valbuf, e1_h.at[pl.ds(base, per_w)], vsem)

    def start_gather(c):
        return pltpu.async_copy(
            t2_h.at[idx2_v.at[c]], rowbuf.at[c % 2], gsem.at[c % 2])

    def start_put(c):
        return pltpu.async_copy(
            rowbuf.at[c % 2], e2_h.at[pl.ds(base + c * _CH, _CH)],
            psem.at[c % 2])

    g_h = [None, None]
    put_h = [None, None]
    g_h[0] = start_gather(0)
    for c in range(nchunk):
        nxt = c + 1
        if nxt < nchunk:
            if put_h[nxt % 2] is not None:
                put_h[nxt % 2].wait()
            g_h[nxt % 2] = start_gather(nxt)
        g_h[c % 2].wait()
        put_h[c % 2] = start_put(c)
    put_h[(nchunk - 1) % 2].wait()
    put_h[(nchunk - 2) % 2].wait()
    vput.wait()

  return _sc_body


@functools.lru_cache(maxsize=2)
def _sc_gather(nb):
  bf = nb * F
  per_w = bf // _NW
  nchunk = per_w // _CH
  return pl.kernel(
    _make_sc_body(per_w, nchunk),
    out_type=(
        jax.ShapeDtypeStruct((bf, D), jnp.float32),
        jax.ShapeDtypeStruct((bf,), jnp.float32),
    ),
    mesh=plsc.VectorSubcoreMesh(core_axis_name="c", subcore_axis_name="s",
                                num_cores=_NC, num_subcores=_NS),
    scratch_types=[
        pltpu.VMEM((nchunk, _CH), jnp.int32),
        pltpu.VMEM((nchunk, _CH), jnp.int32),
        pltpu.VMEM((26000,), jnp.float32),
        pltpu.VMEM((2, _CH, D), jnp.float32),
        pltpu.VMEM((per_w,), jnp.float32),
        pltpu.SemaphoreType.DMA((2,)),
        pltpu.SemaphoreType.DMA((2,)),
        pltpu.SemaphoreType.DMA,
    ],
    compiler_params=pltpu.CompilerParams(needs_layout_passes=False),
  )


def _tc_body(E_ref, e1_ref, W1_ref, b1_ref, W2_ref, b2_ref, W3_ref, b3_ref,
             S_ref, wh_ref, sc_ref, out_ref):
    Es = [E_ref[f].astype(jnp.bfloat16) for f in range(F)]

    acc = b1_ref[...][None, :]
    for f in range(F):
        acc = acc + lax.dot_general(
            Es[f], W1_ref[f],
            (((1,), (0,)), ((), ())), preferred_element_type=jnp.float32)
    h = jnp.maximum(acc, 0.0)
    h = jnp.maximum(
        lax.dot_general(h, W2_ref[...], (((1,), (0,)), ((), ())),
                        preferred_element_type=jnp.float32)
        + b2_ref[...][None, :], 0.0)
    h = jnp.maximum(
        lax.dot_general(h, W3_ref[...], (((1,), (0,)), ((), ())),
                        preferred_element_type=jnp.float32)
        + b3_ref[...][None, :], 0.0)
    deep = lax.dot_general(h, wh_ref[...], (((1,), (0,)), ((), ())),
                           preferred_element_type=jnp.float32)

    M = None
    for f in range(F - 1):
        r = None
        for g in range(f + 1, F):
            t = S_ref[f, g].astype(jnp.bfloat16) * Es[g]
            r = t if r is None else r + t
        m = Es[f] * r
        M = m if M is None else M + m
    pair = jnp.sum(M.astype(jnp.float32), axis=1, keepdims=True)

    first = jnp.sum(e1_ref[...], axis=1, keepdims=True)
    logit = sc_ref[0] * first + pair + deep + sc_ref[1]
    out_ref[...] = jax.nn.sigmoid(logit)


@functools.lru_cache(maxsize=2)
def _tc_call(nb):
  return pl.pallas_call(
    _tc_body,
    grid=(nb // _BLK,),
    in_specs=[
        pl.BlockSpec((F, _BLK, D), lambda i: (0, i, 0)),
        pl.BlockSpec((_BLK, F), lambda i: (i, 0)),
        pl.BlockSpec((F, D, 128), lambda i: (0, 0, 0)),
        pl.BlockSpec((128,), lambda i: (0,)),
        pl.BlockSpec((128, 64), lambda i: (0, 0)),
        pl.BlockSpec((64,), lambda i: (0,)),
        pl.BlockSpec((64, 32), lambda i: (0, 0)),
        pl.BlockSpec((32,), lambda i: (0,)),
        pl.BlockSpec(memory_space=pltpu.SMEM),
        pl.BlockSpec((32, 1), lambda i: (0, 0)),
        pl.BlockSpec(memory_space=pltpu.SMEM),
    ],
    out_specs=pl.BlockSpec((_BLK, 1), lambda i: (i, 0)),
    out_shape=jax.ShapeDtypeStruct((nb, 1), jnp.float32),
  )


_OFFSETS = np.concatenate([[0], np.cumsum([1000] * 26)[:-1]]).astype(np.int32)
_IU, _JU = np.triu_indices(F, k=1)


def kernel(x, table1, table2, W1, b1, W2, b2, W3, b3, fcW, fcb):
    idx = x + jnp.asarray(_OFFSETS)[None, :]
    idxT = idx.T
    t1f = table1.reshape(-1)
    S = jnp.zeros((F, F), jnp.float32).at[_IU, _JU].set(fcW[1:1 + NP, 0])
    wh = fcW[1 + NP:, :]
    sc = jnp.concatenate([fcW[0], fcb])
    W1r = W1.astype(jnp.bfloat16).reshape(F, D, 128)

    gathered = []
    nchunk = F * _NB // _CH // _NW
    for s in range(_NSPLIT):
        lo = s * _NB
        idxfm3 = idxT[:, lo:lo + _NB].reshape(_NW, nchunk, _CH)
        idxb3 = idx[lo:lo + _NB].reshape(_NW, nchunk, _CH)
        gathered.append(_sc_gather(_NB)(table2, t1f, idxfm3, idxb3))
    outs = []
    for e2, e1v in gathered:
        outs.append(_tc_call(_NB)(
            e2.reshape(F, _NB, D), e1v.reshape(_NB, F),
            W1r, b1, W2, b2, W3, b3, S, wh, sc))
    return jnp.concatenate(outs, axis=0)


# speedup vs baseline: 1.0853x; 1.0219x over previous
import functools

import jax
import jax.numpy as jnp
import numpy as np
from jax import lax
from jax.experimental import pallas as pl
from jax.experimental.pallas import tpu as pltpu
from jax.experimental.pallas import tpu_sc as plsc

F = 26
D = 128
B = 16384
NP = F * (F - 1) // 2
DIN = F * D

_NC, _NS = 2, 16
_NW = _NC * _NS
_CH = 128

_BLK = 512
_NSPLIT = 4
_NB = B // _NSPLIT


def _make_sc_body(per_w, nchunk):
  def _sc_body(t2_h, t1_h, idx2_h, idxb2_h, e2_h, e1_h,
               idx2_v, idxb_v, t1_v, rowbuf, valbuf, gsem, psem, vsem):
    wid = lax.axis_index("s") * _NC + lax.axis_index("c")
    base = wid * per_w

    pltpu.sync_copy(idx2_h.at[wid], idx2_v)
    pltpu.sync_copy(idxb2_h.at[wid], idxb_v)

    pltpu.sync_copy(t1_h, t1_v)

    def vbody(c, carry):
        row = idxb_v.at[c]
        for j in range(_CH // 16):
            iv = row[pl.ds(j * 16, 16)]
            valbuf[pl.ds(c * _CH + j * 16, 16)] = plsc.load_gather(t1_v, [iv])
        return carry

    lax.fori_loop(0, nchunk, vbody, 0)
    vput = pltpu.async_copy(valbuf, e1_h.at[pl.ds(base, per_w)], vsem)

    def start_gather(c):
        return pltpu.async_copy(
            t2_h.at[idx2_v.at[c]], rowbuf.at[c % 2], gsem.at[c % 2])

    def start_put(c):
        return pltpu.async_copy(
            rowbuf.at[c % 2], e2_h.at[pl.ds(base + c * _CH, _CH)],
            psem.at[c % 2])

    g_h = [None, None]
    put_h = [None, None]
    g_h[0] = start_gather(0)
    for c in range(nchunk):
        nxt = c + 1
        if nxt < nchunk:
            if put_h[nxt % 2] is not None:
                put_h[nxt % 2].wait()
            g_h[nxt % 2] = start_gather(nxt)
        g_h[c % 2].wait()
        put_h[c % 2] = start_put(c)
    put_h[(nchunk - 1) % 2].wait()
    put_h[(nchunk - 2) % 2].wait()
    vput.wait()

  return _sc_body


@functools.lru_cache(maxsize=2)
def _sc_gather(nb):
  bf = nb * F
  per_w = bf // _NW
  nchunk = per_w // _CH
  return pl.kernel(
    _make_sc_body(per_w, nchunk),
    out_type=(
        jax.ShapeDtypeStruct((bf, D), jnp.float32),
        jax.ShapeDtypeStruct((bf,), jnp.float32),
    ),
    mesh=plsc.VectorSubcoreMesh(core_axis_name="c", subcore_axis_name="s",
                                num_cores=_NC, num_subcores=_NS),
    scratch_types=[
        pltpu.VMEM((nchunk, _CH), jnp.int32),
        pltpu.VMEM((nchunk, _CH), jnp.int32),
        pltpu.VMEM((26000,), jnp.float32),
        pltpu.VMEM((2, _CH, D), jnp.float32),
        pltpu.VMEM((per_w,), jnp.float32),
        pltpu.SemaphoreType.DMA((2,)),
        pltpu.SemaphoreType.DMA((2,)),
        pltpu.SemaphoreType.DMA,
    ],
    compiler_params=pltpu.CompilerParams(needs_layout_passes=False),
  )


def _tc_body(E_ref, e1_ref, W1_ref, b1_ref, W2_ref, b2_ref, W3_ref, b3_ref,
             S_ref, wh_ref, sc_ref, out_ref):
    Es = [E_ref[f].astype(jnp.bfloat16) for f in range(F)]

    acc = b1_ref[...][None, :]
    for f in range(F):
        acc = acc + lax.dot_general(
            Es[f], W1_ref[f],
            (((1,), (0,)), ((), ())), preferred_element_type=jnp.float32)
    h = jnp.maximum(acc, 0.0)
    h = jnp.maximum(
        lax.dot_general(h, W2_ref[...], (((1,), (0,)), ((), ())),
                        preferred_element_type=jnp.float32)
        + b2_ref[...][None, :], 0.0)
    h = jnp.maximum(
        lax.dot_general(h, W3_ref[...], (((1,), (0,)), ((), ())),
                        preferred_element_type=jnp.float32)
        + b3_ref[...][None, :], 0.0)
    deep = lax.dot_general(h, wh_ref[...], (((1,), (0,)), ((), ())),
                           preferred_element_type=jnp.float32)

    M = None
    for f in range(F - 1):
        r = None
        for g in range(f + 1, F):
            t = S_ref[f, g].astype(jnp.bfloat16) * Es[g]
            r = t if r is None else r + t
        m = Es[f] * r
        M = m if M is None else M + m
    pair = jnp.sum(M.astype(jnp.float32), axis=1, keepdims=True)

    first = jnp.sum(e1_ref[...], axis=1, keepdims=True)
    logit = sc_ref[0] * first + pair + deep + sc_ref[1]
    out_ref[...] = jax.nn.sigmoid(logit)


@functools.lru_cache(maxsize=2)
def _tc_call(nb):
  return pl.pallas_call(
    _tc_body,
    grid=(nb // _BLK,),
    in_specs=[
        pl.BlockSpec((F, _BLK, D), lambda i: (0, i, 0)),
        pl.BlockSpec((_BLK, F), lambda i: (i, 0)),
        pl.BlockSpec((F, D, 128), lambda i: (0, 0, 0)),
        pl.BlockSpec((128,), lambda i: (0,)),
        pl.BlockSpec((128, 64), lambda i: (0, 0)),
        pl.BlockSpec((64,), lambda i: (0,)),
        pl.BlockSpec((64, 32), lambda i: (0, 0)),
        pl.BlockSpec((32,), lambda i: (0,)),
        pl.BlockSpec(memory_space=pltpu.SMEM),
        pl.BlockSpec((32, 1), lambda i: (0, 0)),
        pl.BlockSpec(memory_space=pltpu.SMEM),
    ],
    out_specs=pl.BlockSpec((_BLK, 1), lambda i: (i, 0)),
    out_shape=jax.ShapeDtypeStruct((nb, 1), jnp.float32),
  )


_OFFSETS = np.concatenate([[0], np.cumsum([1000] * 26)[:-1]]).astype(np.int32)
_IU, _JU = np.triu_indices(F, k=1)


def kernel(x, table1, table2, W1, b1, W2, b2, W3, b3, fcW, fcb):
    idx = x + jnp.asarray(_OFFSETS)[None, :]
    idxT = idx.T
    t1f = table1.reshape(-1)
    S = jnp.zeros((F, F), jnp.float32).at[_IU, _JU].set(fcW[1:1 + NP, 0])
    wh = fcW[1 + NP:, :]
    sc = jnp.concatenate([fcW[0], fcb])
    W1r = W1.astype(jnp.bfloat16).reshape(F, D, 128)

    gathered = []
    nchunk = F * _NB // _CH // _NW
    for s in range(_NSPLIT):
        lo = s * _NB
        idxfm3 = idxT[:, lo:lo + _NB].reshape(_NW, nchunk, _CH)
        idxb3 = idx[lo:lo + _NB].reshape(_NW, nchunk, _CH)
        gathered.append(_sc_gather(_NB)(table2, t1f, idxfm3, idxb3))
    outs = []
    for e2, e1v in gathered:
        outs.append(_tc_call(_NB)(
            e2.reshape(F, _NB, D), e1v.reshape(_NB, F),
            W1r, b1, W2, b2, W3, b3, S, wh, sc))
    return jnp.concatenate(outs, axis=0)
